# baseline (speedup 1.0000x reference)
import functools

import jax
import jax.numpy as jnp
from jax import lax
from jax.experimental import pallas as pl
from jax.experimental.pallas import tpu as pltpu
from jax.experimental.pallas import tpu_sc as plsc

N = 10000
E = 320000
H = 8
D = 16
EF = 64
NUM_ETYPES = 5
NUM_NTYPES = 4
NEG_SLOPE = 0.2
HD = H * D

NC = 2
NS = 16
NW = NC * NS
EPW = E // NW
CE = 80
NCHUNK = EPW // CE
NP = 10240
RPT = NP // NS
CR = 40
NCHUNK_N = N // CR

_mesh = plsc.VectorSubcoreMesh(core_axis_name="c", subcore_axis_name="s")
_f32 = jnp.float32
_sc_params = pltpu.CompilerParams(needs_layout_passes=False,
                                  use_tc_tiling_on_sc=False)


def _iota16():
    return lax.broadcasted_iota(jnp.int32, (16,), 0)


def _perm(v, idx):
    return v.at[idx].get(mode="promise_in_bounds")


def _splat(j):
    return jnp.full((16,), j, jnp.int32)


RB = 1000
NB = N // RB


def _k0_body(feat_ref, nt_ref, fcp_ref, w_ref, eemb_ref, fcew_ref, v2_ref,
             fs_ref, elr_ref, eetab_ref):
    i = pl.program_id(0)
    nt = nt_ref[0, 0, :]
    onehot = (nt[:, None] == lax.broadcasted_iota(jnp.int32, (RB, 8), 1))
    scale = jax.lax.dot_general(
        onehot.astype(_f32), fcp_ref[...], (((1,), (0,)), ((), ())),
        precision=lax.Precision.HIGHEST, preferred_element_type=_f32)
    fs = feat_ref[...] * scale
    fs_ref[...] = fs
    elr_ref[...] = jax.lax.dot_general(
        fs, w_ref[...], (((1,), (0,)), ((), ())),
        precision=lax.Precision.HIGHEST, preferred_element_type=_f32)

    @pl.when(i == 0)
    def _():
        m = jax.lax.dot_general(
            eemb_ref[...], fcew_ref[...], (((1,), (1,)), ((), ())),
            precision=lax.Precision.HIGHEST, preferred_element_type=_f32)
        eetab_ref[...] = jax.lax.dot_general(
            m, v2_ref[...], (((1,), (0,)), ((), ())),
            precision=lax.Precision.HIGHEST, preferred_element_type=_f32)


_k0 = pl.pallas_call(
    _k0_body,
    grid=(NB,),
    in_specs=[
        pl.BlockSpec((RB, HD), lambda i: (i, 0)),
        pl.BlockSpec((1, 1, RB), lambda i: (i, 0, 0)),
        pl.BlockSpec((8, HD), lambda i: (0, 0)),
        pl.BlockSpec((HD, 16), lambda i: (0, 0)),
        pl.BlockSpec((8, EF), lambda i: (0, 0)),
        pl.BlockSpec((H * EF, EF), lambda i: (0, 0)),
        pl.BlockSpec((H * EF, 16), lambda i: (0, 0)),
    ],
    out_specs=[
        pl.BlockSpec((RB, HD), lambda i: (i, 0)),
        pl.BlockSpec((RB, 16), lambda i: (i, 0)),
        pl.BlockSpec((8, 16), lambda i: (0, 0)),
    ],
    out_shape=[
        jax.ShapeDtypeStruct((N, HD), _f32),
        jax.ShapeDtypeStruct((N, 16), _f32),
        jax.ShapeDtypeStruct((8, 16), _f32),
    ],
)


def _k1_body(sde_hbm, elr_hbm, fs_hbm, eetab_hbm, z16_hbm,
             z128_hbm, ex_hbm, denp_hbm, rstp_hbm,
             sdeb, dsc, elrS, elrD, featb, exb, eetv, den_sp,
             rst_sp, is0, is1, gsS0, gsS1, gsD0, gsD1, gsF0, gsF1,
             ssE0, ssE1, ssD0, ssD1, ssR0, ssR1):
    c = lax.axis_index("c")
    s = lax.axis_index("s")
    wid = c * NS + s
    iota16 = _iota16()
    idx_hi = (iota16 & 7) | 8
    low = iota16 < 8

    pltpu.sync_copy(z16_hbm, den_sp.at[pl.ds(s * RPT, RPT)])
    pltpu.sync_copy(z128_hbm, rst_sp.at[pl.ds(s * RPT, RPT)])
    pltpu.sync_copy(eetab_hbm, eetv)
    plsc.subcore_barrier()

    isem = (is0, is1)
    gsS = (gsS0, gsS1)
    gsD = (gsD0, gsD1)
    gsF = (gsF0, gsF1)
    ssE = (ssE0, ssE1)
    ssDn = (ssD0, ssD1)
    ssR = (ssR0, ssR1)
    ebase0 = wid * EPW

    def fire_idx(cur, b):
        pltpu.async_copy(sde_hbm.at[wid, cur], sdeb.at[b], isem[b])

    def wait_idx(cur, b):
        pltpu.make_async_copy(sde_hbm.at[wid, cur], sdeb.at[b],
                              isem[b]).wait()

    def fire_gathers(cur, b):
        pltpu.async_copy(elr_hbm.at[sdeb.at[b, 0]], elrS.at[b], gsS[b])
        pltpu.async_copy(elr_hbm.at[sdeb.at[b, 1]], elrD.at[b], gsD[b])
        pltpu.async_copy(fs_hbm.at[sdeb.at[b, 0]], featb.at[b], gsF[b])

    def wait_gathers(cur, b):
        pltpu.make_async_copy(elr_hbm.at[sdeb.at[b, 0]], elrS.at[b],
                              gsS[b]).wait()
        pltpu.make_async_copy(elr_hbm.at[sdeb.at[b, 1]], elrD.at[b],
                              gsD[b]).wait()
        pltpu.make_async_copy(fs_hbm.at[sdeb.at[b, 0]], featb.at[b],
                              gsF[b]).wait()

    def fire_scatters(cur, b):
        pltpu.async_copy(exb.at[b], ex_hbm.at[pl.ds(ebase0 + cur * CE, CE)],
                         ssE[b])
        pltpu.async_copy(exb.at[b], den_sp.at[dsc.at[b]], ssDn[b],
                         add=True)
        pltpu.async_copy(featb.at[b], rst_sp.at[dsc.at[b]], ssR[b],
                         add=True)

    def wait_scatters(cur, b):
        pltpu.make_async_copy(exb.at[b],
                              ex_hbm.at[pl.ds(ebase0 + cur * CE, CE)],
                              ssE[b]).wait()
        pltpu.make_async_copy(exb.at[b], den_sp.at[dsc.at[b]],
                              ssDn[b]).wait()
        pltpu.make_async_copy(featb.at[b], rst_sp.at[dsc.at[b]],
                              ssR[b]).wait()

    def compute(cur, b):
        for g in range(CE // 16):
            g16 = g * 16
            efv = sdeb[b, 2, pl.ds(g16, 16)]
            combs = [jnp.where(low, elrS[b, g16 + jj], elrD[b, g16 + jj])
                     for jj in range(16)]
            esums = [cmb + _perm(cmb, idx_hi) for cmb in combs]
            vees = [plsc.load_gather(eetv,
                                     [_perm(efv, _splat(jj)) * 16 + iota16])
                    for jj in range(16)]
            exs = []
            for jj in range(16):
                e = esums[jj] + vees[jj]
                e = jnp.maximum(e, NEG_SLOPE * e)
                exs.append(jnp.exp(e))
            for jj in range(16):
                exb[b, g16 + jj] = exs[jj]
            for h in range(H):
                for jj in range(16):
                    bb = _perm(exs[jj], _splat(h))
                    featb[b, g16 + jj, pl.ds(h * 16, 16)] = (
                        featb[b, g16 + jj, pl.ds(h * 16, 16)] * bb)
        for g in range(CE // 16):
            dsc[b, pl.ds(g * 16, 16)] = sdeb[b, 1, pl.ds(g * 16, 16)]

    fire_idx(0, 0)
    wait_idx(0, 0)
    fire_gathers(0, 0)
    fire_idx(1, 1)

    @pl.loop(0, NCHUNK + 1, step=2)
    def _pair(i):
        for b in range(2):
            cur = i + b

            @pl.when(cur < NCHUNK)
            def _():
                nxt = cur + 1

                @pl.when(nxt < NCHUNK)
                def _():
                    @pl.when(cur >= 1)
                    def _():
                        wait_scatters(cur - 1, 1 - b)

                    wait_idx(nxt, 1 - b)
                    fire_gathers(nxt, 1 - b)

                wait_gathers(cur, b)
                compute(cur, b)
                fire_scatters(cur, b)

                @pl.when(cur + 2 < NCHUNK)
                def _():
                    fire_idx(cur + 2, b)

    wait_scatters(NCHUNK - 1, (NCHUNK - 1) % 2)
    wait_scatters(NCHUNK - 2, (NCHUNK - 2) % 2)
    plsc.subcore_barrier()
    pltpu.sync_copy(den_sp.at[pl.ds(s * RPT, RPT)],
                    denp_hbm.at[pl.ds(c * NP + s * RPT, RPT)])
    pltpu.sync_copy(rst_sp.at[pl.ds(s * RPT, RPT)],
                    rstp_hbm.at[pl.ds(c * NP + s * RPT, RPT)])


_k1 = functools.partial(
    pl.kernel,
    out_type=[
        jax.ShapeDtypeStruct((E, 16), _f32),
        jax.ShapeDtypeStruct((NC * NP, 16), _f32),
        jax.ShapeDtypeStruct((NC * NP, HD), _f32),
    ],
    mesh=_mesh,
    compiler_params=_sc_params,
    scratch_types=[
        pltpu.VMEM((2, 3, CE), jnp.int32),
        pltpu.VMEM((2, CE), jnp.int32),
        pltpu.VMEM((2, CE, 16), _f32),
        pltpu.VMEM((2, CE, 16), _f32),
        pltpu.VMEM((2, CE, HD), _f32),
        pltpu.VMEM((2, CE, 16), _f32),
        pltpu.VMEM((8 * 16,), _f32),
        pltpu.VMEM_SHARED((NP, 16), _f32),
        pltpu.VMEM_SHARED((NP, HD), _f32),
    ] + [pltpu.SemaphoreType.DMA] * 14,
)(_k1_body)


def _k3_body(dst_hbm, ex_hbm, denp_hbm, rstp_hbm, a_hbm, rst_hbm,
             dstb, dst2b, exb, denb0, denb1, ab, d0b, d1b, u0b, u1b, rb,
             is0, is1, ge0, ge1, gd0, gd1, gq0, gq1, sa0, sa1):
    c = lax.axis_index("c")
    s = lax.axis_index("s")
    wid = c * NS + s
    iota16 = _iota16()
    idx_lo = iota16 & 7
    low = iota16 < 8
    ebase0 = wid * EPW
    isem = (is0, is1)
    gse = (ge0, ge1)
    gsd = (gd0, gd1)
    gsq = (gq0, gq1)
    ssa = (sa0, sa1)

    def fire_idx(cur, b):
        pltpu.async_copy(dst_hbm.at[wid, cur], dstb.at[b], isem[b])

    def wait_idx(cur, b):
        pltpu.make_async_copy(dst_hbm.at[wid, cur], dstb.at[b],
                              isem[b]).wait()

    def fire_loads(cur, b):
        for g in range(CE // 16):
            dst2b[b, pl.ds(g * 16, 16)] = dstb[b, pl.ds(g * 16, 16)] + NP
        pltpu.async_copy(ex_hbm.at[pl.ds(ebase0 + cur * CE, CE)], exb.at[b],
                         gse[b])
        pltpu.async_copy(denp_hbm.at[dstb.at[b]], denb0.at[b], gsd[b])
        pltpu.async_copy(denp_hbm.at[dst2b.at[b]], denb1.at[b], gsq[b])

    def wait_loads(cur, b):
        pltpu.make_async_copy(ex_hbm.at[pl.ds(ebase0 + cur * CE, CE)],
                              exb.at[b], gse[b]).wait()
        pltpu.make_async_copy(denp_hbm.at[dstb.at[b]], denb0.at[b],
                              gsd[b]).wait()
        pltpu.make_async_copy(denp_hbm.at[dst2b.at[b]], denb1.at[b],
                              gsq[b]).wait()

    def fire_store(cur, b):
        pltpu.async_copy(ab.at[b],
                         a_hbm.at[pl.ds((ebase0 + cur * CE) * 8, CE * 8)],
                         ssa[b])

    def wait_store(cur, b):
        pltpu.make_async_copy(ab.at[b],
                              a_hbm.at[pl.ds((ebase0 + cur * CE) * 8,
                                             CE * 8)],
                              ssa[b]).wait()

    def compute(cur, b):
        for j2 in range(CE // 2):
            j = 2 * j2
            a0 = exb[b, j] / (denb0[b, j] + denb1[b, j])
            a1 = exb[b, j + 1] / (denb0[b, j + 1] + denb1[b, j + 1])
            ab[b, pl.ds(j * 8, 16)] = jnp.where(low, a0, _perm(a1, idx_lo))

    fire_idx(0, 0)
    wait_idx(0, 0)
    fire_loads(0, 0)
    fire_idx(1, 1)

    @pl.loop(0, NCHUNK + 1, step=2)
    def _pair(i):
        for b in range(2):
            cur = i + b

            @pl.when(cur < NCHUNK)
            def _():
                nxt = cur + 1

                @pl.when(nxt < NCHUNK)
                def _():
                    @pl.when(cur >= 1)
                    def _():
                        wait_store(cur - 1, 1 - b)

                    wait_idx(nxt, 1 - b)
                    fire_loads(nxt, 1 - b)

                wait_loads(cur, b)
                compute(cur, b)
                fire_store(cur, b)

                @pl.when(cur + 2 < NCHUNK)
                def _():
                    fire_idx(cur + 2, b)

    wait_store(NCHUNK - 1, (NCHUNK - 1) % 2)
    wait_store(NCHUNK - 2, (NCHUNK - 2) % 2)

    @pl.loop(0, (NCHUNK_N + NW - 1) // NW)
    def _blk(k):
        cid = wid + k * NW

        @pl.when(cid < NCHUNK_N)
        def _():
            r0 = cid * CR
            pltpu.sync_copy(denp_hbm.at[pl.ds(r0, CR)], d0b)
            pltpu.sync_copy(denp_hbm.at[pl.ds(NP + r0, CR)], d1b)
            pltpu.sync_copy(rstp_hbm.at[pl.ds(r0, CR)], u0b)
            pltpu.sync_copy(rstp_hbm.at[pl.ds(NP + r0, CR)], u1b)
            for r in range(CR):
                dv = d0b[r] + d1b[r]
                for h in range(H):
                    db = _perm(dv, _splat(h))
                    uv = u0b[r, pl.ds(h * 16, 16)] + u1b[r, pl.ds(h * 16, 16)]
                    rb[r, pl.ds(h * 16, 16)] = jnp.where(db > 0.0, uv / db,
                                                         0.0)
            pltpu.sync_copy(rb, rst_hbm.at[pl.ds(r0, CR)])


_k3 = functools.partial(
    pl.kernel,
    out_type=[
        jax.ShapeDtypeStruct((E * 8,), _f32),
        jax.ShapeDtypeStruct((N, HD), _f32),
    ],
    mesh=_mesh,
    compiler_params=_sc_params,
    scratch_types=[
        pltpu.VMEM((2, CE), jnp.int32),
        pltpu.VMEM((2, CE), jnp.int32),
        pltpu.VMEM((2, CE, 16), _f32),
        pltpu.VMEM((2, CE, 16), _f32),
        pltpu.VMEM((2, CE, 16), _f32),
        pltpu.VMEM((2, CE * 8), _f32),
        pltpu.VMEM((CR, 16), _f32),
        pltpu.VMEM((CR, 16), _f32),
        pltpu.VMEM((CR, HD), _f32),
        pltpu.VMEM((CR, HD), _f32),
        pltpu.VMEM((CR, HD), _f32),
    ] + [pltpu.SemaphoreType.DMA] * 10,
)(_k3_body)


def kernel(feat, edge_index, e_feat, node_types, fc, edge_emb, fc_e_w,
           attn_l, attn_r, attn_e):
    al = attn_l.reshape(HD)
    ar = attn_r.reshape(HD)
    sel = (jnp.arange(HD)[:, None] // D
           == jnp.arange(H)[None, :]).astype(_f32)
    w = jnp.concatenate([al[:, None] * sel, ar[:, None] * sel], axis=1)
    ae = attn_e.reshape(H * EF)
    sel2 = (jnp.arange(H * EF)[:, None] // EF
            == jnp.arange(H)[None, :]).astype(_f32)
    v2 = jnp.concatenate([ae[:, None] * sel2, ae[:, None] * sel2], axis=1)
    fcp = jnp.zeros((8, HD), _f32).at[:NUM_NTYPES].set(fc.reshape(NUM_NTYPES, HD))
    eemb = jnp.zeros((8, EF), _f32).at[:NUM_ETYPES].set(edge_emb)
    nt3 = node_types.reshape(NB, 1, RB)

    fs, elr, eetab = _k0(feat, nt3, fcp, w, eemb, fc_e_w, v2)

    src = edge_index[0]
    dst = edge_index[1]
    z16 = jnp.zeros((RPT, 16), _f32)
    z128 = jnp.zeros((RPT, HD), _f32)
    sde = jnp.stack([src.reshape(NW, NCHUNK, CE),
                     dst.reshape(NW, NCHUNK, CE),
                     e_feat.reshape(NW, NCHUNK, CE)], axis=2)
    exh, denp, rstp = _k1(sde, elr, fs, eetab.reshape(8 * 16), z16, z128)
    a, rst = _k3(dst.reshape(NW, NCHUNK, CE), exh, denp, rstp)
    return rst.reshape(N, H, D), a.reshape(E, H, 1)


# speedup vs baseline: 1.0189x; 1.0189x over previous
import functools

import jax
import jax.numpy as jnp
from jax import lax
from jax.experimental import pallas as pl
from jax.experimental.pallas import tpu as pltpu
from jax.experimental.pallas import tpu_sc as plsc

N = 10000
E = 320000
H = 8
D = 16
EF = 64
NUM_ETYPES = 5
NUM_NTYPES = 4
NEG_SLOPE = 0.2
HD = H * D

NC = 2
NS = 16
NW = NC * NS
EPW = E // NW
CE = 80
NCHUNK = EPW // CE
NP = 10240
RPT = NP // NS
CR = 40
NCHUNK_N = N // CR

_mesh = plsc.VectorSubcoreMesh(core_axis_name="c", subcore_axis_name="s")
_f32 = jnp.float32
_sc_params = pltpu.CompilerParams(needs_layout_passes=False,
                                  use_tc_tiling_on_sc=False)


def _iota16():
    return lax.broadcasted_iota(jnp.int32, (16,), 0)


def _perm(v, idx):
    return v.at[idx].get(mode="promise_in_bounds")


def _splat(j):
    return jnp.full((16,), j, jnp.int32)


RB = 1000
NB = N // RB


def _k0_body(feat_ref, nt_ref, fcp_ref, w_ref, eemb_ref, fcew_ref, v2_ref,
             fs_ref, elr_ref, eetab_ref):
    i = pl.program_id(0)
    nt = nt_ref[0, 0, :]
    onehot = (nt[:, None] == lax.broadcasted_iota(jnp.int32, (RB, 8), 1))
    scale = jax.lax.dot_general(
        onehot.astype(_f32), fcp_ref[...], (((1,), (0,)), ((), ())),
        precision=lax.Precision.HIGHEST, preferred_element_type=_f32)
    fs = feat_ref[...] * scale
    fs_ref[...] = fs
    elr_ref[...] = jax.lax.dot_general(
        fs, w_ref[...], (((1,), (0,)), ((), ())),
        precision=lax.Precision.HIGHEST, preferred_element_type=_f32)

    @pl.when(i == 0)
    def _():
        m = jax.lax.dot_general(
            eemb_ref[...], fcew_ref[...], (((1,), (1,)), ((), ())),
            precision=lax.Precision.HIGHEST, preferred_element_type=_f32)
        eetab_ref[...] = jax.lax.dot_general(
            m, v2_ref[...], (((1,), (0,)), ((), ())),
            precision=lax.Precision.HIGHEST, preferred_element_type=_f32)


_k0 = pl.pallas_call(
    _k0_body,
    grid=(NB,),
    in_specs=[
        pl.BlockSpec((RB, HD), lambda i: (i, 0)),
        pl.BlockSpec((1, 1, RB), lambda i: (i, 0, 0)),
        pl.BlockSpec((8, HD), lambda i: (0, 0)),
        pl.BlockSpec((HD, 16), lambda i: (0, 0)),
        pl.BlockSpec((8, EF), lambda i: (0, 0)),
        pl.BlockSpec((H * EF, EF), lambda i: (0, 0)),
        pl.BlockSpec((H * EF, 16), lambda i: (0, 0)),
    ],
    out_specs=[
        pl.BlockSpec((RB, HD), lambda i: (i, 0)),
        pl.BlockSpec((RB, 16), lambda i: (i, 0)),
        pl.BlockSpec((8, 16), lambda i: (0, 0)),
    ],
    out_shape=[
        jax.ShapeDtypeStruct((N, HD), _f32),
        jax.ShapeDtypeStruct((N, 16), _f32),
        jax.ShapeDtypeStruct((8, 16), _f32),
    ],
)


def _k1_body(sde_hbm, elr_hbm, fs_hbm, eetab_hbm, z16_hbm,
             z128_hbm, ex_hbm, denp_hbm, rstp_hbm,
             sdeb, dsc, elrS, elrD, featb, exb, eetv, den_sp,
             rst_sp, is0, is1, gsS0, gsS1, gsD0, gsD1, gsF0, gsF1,
             ssE0, ssE1, ssD0, ssD1, ssR0, ssR1):
    c = lax.axis_index("c")
    s = lax.axis_index("s")
    wid = c * NS + s
    iota16 = _iota16()
    idx_hi = (iota16 & 7) | 8
    low = iota16 < 8

    pltpu.sync_copy(z16_hbm, den_sp.at[pl.ds(s * RPT, RPT)])
    pltpu.sync_copy(z128_hbm, rst_sp.at[pl.ds(s * RPT, RPT)])
    pltpu.sync_copy(eetab_hbm, eetv)
    plsc.subcore_barrier()

    isem = (is0, is1)
    gsS = (gsS0, gsS1)
    gsD = (gsD0, gsD1)
    gsF = (gsF0, gsF1)
    ssE = (ssE0, ssE1)
    ssDn = (ssD0, ssD1)
    ssR = (ssR0, ssR1)
    ebase0 = wid * EPW

    def fire_idx(cur, b):
        pltpu.async_copy(sde_hbm.at[wid, cur], sdeb.at[b], isem[b])

    def wait_idx(cur, b):
        pltpu.make_async_copy(sde_hbm.at[wid, cur], sdeb.at[b],
                              isem[b]).wait()

    def fire_gathers(cur, b):
        pltpu.async_copy(elr_hbm.at[sdeb.at[b, 0]], elrS.at[b], gsS[b])
        pltpu.async_copy(elr_hbm.at[sdeb.at[b, 1]], elrD.at[b], gsD[b])
        pltpu.async_copy(fs_hbm.at[sdeb.at[b, 0]], featb.at[b], gsF[b])

    def wait_gathers(cur, b):
        pltpu.make_async_copy(elr_hbm.at[sdeb.at[b, 0]], elrS.at[b],
                              gsS[b]).wait()
        pltpu.make_async_copy(elr_hbm.at[sdeb.at[b, 1]], elrD.at[b],
                              gsD[b]).wait()
        pltpu.make_async_copy(fs_hbm.at[sdeb.at[b, 0]], featb.at[b],
                              gsF[b]).wait()

    def fire_scatters(cur, b):
        pltpu.async_copy(exb.at[b], ex_hbm.at[pl.ds(ebase0 + cur * CE, CE)],
                         ssE[b])
        pltpu.async_copy(exb.at[b], den_sp.at[dsc.at[b]], ssDn[b],
                         add=True)
        pltpu.async_copy(featb.at[b], rst_sp.at[dsc.at[b]], ssR[b],
                         add=True)

    def wait_scatters(cur, b):
        pltpu.make_async_copy(exb.at[b],
                              ex_hbm.at[pl.ds(ebase0 + cur * CE, CE)],
                              ssE[b]).wait()
        pltpu.make_async_copy(exb.at[b], den_sp.at[dsc.at[b]],
                              ssDn[b]).wait()
        pltpu.make_async_copy(featb.at[b], rst_sp.at[dsc.at[b]],
                              ssR[b]).wait()

    def compute(cur, b):
        for g in range(CE // 16):
            g16 = g * 16
            efv = sdeb[b, 2, pl.ds(g16, 16)]
            combs = [jnp.where(low, elrS[b, g16 + jj], elrD[b, g16 + jj])
                     for jj in range(16)]
            esums = [cmb + _perm(cmb, idx_hi) for cmb in combs]
            vees = [plsc.load_gather(eetv,
                                     [_perm(efv, _splat(jj)) * 16 + iota16])
                    for jj in range(16)]
            exs = []
            for jj in range(16):
                e = esums[jj] + vees[jj]
                e = jnp.maximum(e, NEG_SLOPE * e)
                exs.append(jnp.exp(e))
            for jj in range(16):
                exb[b, g16 + jj] = exs[jj]
            for h in range(H):
                for jj in range(16):
                    bb = _perm(exs[jj], _splat(h))
                    featb[b, g16 + jj, pl.ds(h * 16, 16)] = (
                        featb[b, g16 + jj, pl.ds(h * 16, 16)] * bb)
        for g in range(CE // 16):
            dsc[b, pl.ds(g * 16, 16)] = sdeb[b, 1, pl.ds(g * 16, 16)]

    fire_idx(0, 0)
    wait_idx(0, 0)
    fire_gathers(0, 0)
    fire_idx(1, 1)

    @pl.loop(0, NCHUNK + 1, step=2)
    def _pair(i):
        for b in range(2):
            cur = i + b

            @pl.when(cur < NCHUNK)
            def _():
                nxt = cur + 1

                @pl.when(nxt < NCHUNK)
                def _():
                    @pl.when(cur >= 1)
                    def _():
                        wait_scatters(cur - 1, 1 - b)

                    wait_idx(nxt, 1 - b)
                    fire_gathers(nxt, 1 - b)

                wait_gathers(cur, b)
                compute(cur, b)
                fire_scatters(cur, b)

                @pl.when(cur + 2 < NCHUNK)
                def _():
                    fire_idx(cur + 2, b)

    wait_scatters(NCHUNK - 1, (NCHUNK - 1) % 2)
    wait_scatters(NCHUNK - 2, (NCHUNK - 2) % 2)
    plsc.subcore_barrier()
    pltpu.sync_copy(den_sp.at[pl.ds(s * RPT, RPT)],
                    denp_hbm.at[pl.ds(c * NP + s * RPT, RPT)])
    pltpu.sync_copy(rst_sp.at[pl.ds(s * RPT, RPT)],
                    rstp_hbm.at[pl.ds(c * NP + s * RPT, RPT)])


_k1 = functools.partial(
    pl.kernel,
    out_type=[
        jax.ShapeDtypeStruct((E, 16), _f32),
        jax.ShapeDtypeStruct((NC * NP, 16), _f32),
        jax.ShapeDtypeStruct((NC * NP, HD), _f32),
    ],
    mesh=_mesh,
    compiler_params=_sc_params,
    scratch_types=[
        pltpu.VMEM((2, 3, CE), jnp.int32),
        pltpu.VMEM((2, CE), jnp.int32),
        pltpu.VMEM((2, CE, 16), _f32),
        pltpu.VMEM((2, CE, 16), _f32),
        pltpu.VMEM((2, CE, HD), _f32),
        pltpu.VMEM((2, CE, 16), _f32),
        pltpu.VMEM((8 * 16,), _f32),
        pltpu.VMEM_SHARED((NP, 16), _f32),
        pltpu.VMEM_SHARED((NP, HD), _f32),
    ] + [pltpu.SemaphoreType.DMA] * 14,
)(_k1_body)


def _k2_body(denp_hbm, rstp_hbm, deng_hbm, rst_hbm,
             d0b, d1b, u0b, u1b, dgb, rb):
    c = lax.axis_index("c")
    s = lax.axis_index("s")
    wid = c * NS + s

    @pl.loop(0, (NCHUNK_N + NW - 1) // NW)
    def _blk(k):
        cid = wid + k * NW

        @pl.when(cid < NCHUNK_N)
        def _():
            r0 = cid * CR
            pltpu.sync_copy(denp_hbm.at[pl.ds(r0, CR)], d0b)
            pltpu.sync_copy(denp_hbm.at[pl.ds(NP + r0, CR)], d1b)
            pltpu.sync_copy(rstp_hbm.at[pl.ds(r0, CR)], u0b)
            pltpu.sync_copy(rstp_hbm.at[pl.ds(NP + r0, CR)], u1b)
            for r in range(CR):
                dv = d0b[r] + d1b[r]
                dgb[r] = dv
                for h in range(H):
                    db = _perm(dv, _splat(h))
                    uv = u0b[r, pl.ds(h * 16, 16)] + u1b[r, pl.ds(h * 16, 16)]
                    rb[r, pl.ds(h * 16, 16)] = jnp.where(db > 0.0, uv / db, 0.0)
            pltpu.sync_copy(dgb, deng_hbm.at[pl.ds(r0, CR)])
            pltpu.sync_copy(rb, rst_hbm.at[pl.ds(r0, CR)])


_k2 = functools.partial(
    pl.kernel,
    out_type=[
        jax.ShapeDtypeStruct((N, 16), _f32),
        jax.ShapeDtypeStruct((N, HD), _f32),
    ],
    mesh=_mesh,
    compiler_params=_sc_params,
    scratch_types=[
        pltpu.VMEM((CR, 16), _f32),
        pltpu.VMEM((CR, 16), _f32),
        pltpu.VMEM((CR, HD), _f32),
        pltpu.VMEM((CR, HD), _f32),
        pltpu.VMEM((CR, 16), _f32),
        pltpu.VMEM((CR, HD), _f32),
    ],
)(_k2_body)


def _k3_body(dst_hbm, ex_hbm, deng_hbm, a_hbm, dstb, exb, denb, ab,
             is0, is1, ge0, ge1, gd0, gd1, sa0, sa1):
    c = lax.axis_index("c")
    s = lax.axis_index("s")
    wid = c * NS + s
    iota16 = _iota16()
    idx_lo = iota16 & 7
    low = iota16 < 8
    ebase0 = wid * EPW
    isem = (is0, is1)
    gse = (ge0, ge1)
    gsd = (gd0, gd1)
    ssa = (sa0, sa1)

    def fire_idx(cur, b):
        pltpu.async_copy(dst_hbm.at[wid, cur], dstb.at[b], isem[b])

    def wait_idx(cur, b):
        pltpu.make_async_copy(dst_hbm.at[wid, cur], dstb.at[b],
                              isem[b]).wait()

    def fire_loads(cur, b):
        pltpu.async_copy(ex_hbm.at[pl.ds(ebase0 + cur * CE, CE)], exb.at[b],
                         gse[b])
        pltpu.async_copy(deng_hbm.at[dstb.at[b]], denb.at[b], gsd[b])

    def wait_loads(cur, b):
        pltpu.make_async_copy(ex_hbm.at[pl.ds(ebase0 + cur * CE, CE)],
                              exb.at[b], gse[b]).wait()
        pltpu.make_async_copy(deng_hbm.at[dstb.at[b]], denb.at[b],
                              gsd[b]).wait()

    def fire_store(cur, b):
        pltpu.async_copy(ab.at[b],
                         a_hbm.at[pl.ds((ebase0 + cur * CE) * 8, CE * 8)],
                         ssa[b])

    def wait_store(cur, b):
        pltpu.make_async_copy(ab.at[b],
                              a_hbm.at[pl.ds((ebase0 + cur * CE) * 8,
                                             CE * 8)],
                              ssa[b]).wait()

    def compute(cur, b):
        for j2 in range(CE // 2):
            j = 2 * j2
            a0 = exb[b, j] / denb[b, j]
            a1 = exb[b, j + 1] / denb[b, j + 1]
            ab[b, pl.ds(j * 8, 16)] = jnp.where(low, a0, _perm(a1, idx_lo))

    fire_idx(0, 0)
    wait_idx(0, 0)
    fire_loads(0, 0)
    fire_idx(1, 1)

    @pl.loop(0, NCHUNK + 1, step=2)
    def _pair(i):
        for b in range(2):
            cur = i + b

            @pl.when(cur < NCHUNK)
            def _():
                nxt = cur + 1

                @pl.when(nxt < NCHUNK)
                def _():
                    @pl.when(cur >= 1)
                    def _():
                        wait_store(cur - 1, 1 - b)

                    wait_idx(nxt, 1 - b)
                    fire_loads(nxt, 1 - b)

                wait_loads(cur, b)
                compute(cur, b)
                fire_store(cur, b)

                @pl.when(cur + 2 < NCHUNK)
                def _():
                    fire_idx(cur + 2, b)

    wait_store(NCHUNK - 1, (NCHUNK - 1) % 2)
    wait_store(NCHUNK - 2, (NCHUNK - 2) % 2)


_k3 = functools.partial(
    pl.kernel,
    out_type=jax.ShapeDtypeStruct((E * 8,), _f32),
    mesh=_mesh,
    compiler_params=_sc_params,
    scratch_types=[
        pltpu.VMEM((2, CE), jnp.int32),
        pltpu.VMEM((2, CE, 16), _f32),
        pltpu.VMEM((2, CE, 16), _f32),
        pltpu.VMEM((2, CE * 8), _f32),
    ] + [pltpu.SemaphoreType.DMA] * 8,
)(_k3_body)


def kernel(feat, edge_index, e_feat, node_types, fc, edge_emb, fc_e_w,
           attn_l, attn_r, attn_e):
    al = attn_l.reshape(HD)
    ar = attn_r.reshape(HD)
    sel = (jnp.arange(HD)[:, None] // D
           == jnp.arange(H)[None, :]).astype(_f32)
    w = jnp.concatenate([al[:, None] * sel, ar[:, None] * sel], axis=1)
    ae = attn_e.reshape(H * EF)
    sel2 = (jnp.arange(H * EF)[:, None] // EF
            == jnp.arange(H)[None, :]).astype(_f32)
    v2 = jnp.concatenate([ae[:, None] * sel2, ae[:, None] * sel2], axis=1)
    fcp = jnp.zeros((8, HD), _f32).at[:NUM_NTYPES].set(fc.reshape(NUM_NTYPES, HD))
    eemb = jnp.zeros((8, EF), _f32).at[:NUM_ETYPES].set(edge_emb)
    nt3 = node_types.reshape(NB, 1, RB)

    fs, elr, eetab = _k0(feat, nt3, fcp, w, eemb, fc_e_w, v2)

    src = edge_index[0]
    dst = edge_index[1]
    z16 = jnp.zeros((RPT, 16), _f32)
    z128 = jnp.zeros((RPT, HD), _f32)
    sde = jnp.stack([src.reshape(NW, NCHUNK, CE),
                     dst.reshape(NW, NCHUNK, CE),
                     e_feat.reshape(NW, NCHUNK, CE)], axis=2)
    exh, denp, rstp = _k1(sde, elr, fs, eetab.reshape(8 * 16), z16, z128)
    deng, rst = _k2(denp, rstp)
    a = _k3(dst.reshape(NW, NCHUNK, CE), exh, deng)
    return rst.reshape(N, H, D), a.reshape(E, H, 1)
